# baseline (speedup 1.0000x reference)
import functools

import jax
import jax.numpy as jnp
from jax import lax
from jax.experimental import pallas as pl
from jax.experimental.pallas import tpu as pltpu
from jax.experimental.pallas import tpu_sc as plsc

B, N, D = 1024, 32, 64
NW = 32
BPW = B // NW
P = N * (N - 1) // 2
NQ = D // 16


def _edge_body(emb_hbm, out_hbm, emb_v, out_v):
    wid = lax.axis_index("s") * 2 + lax.axis_index("c")
    base = wid * BPW
    pltpu.sync_copy(emb_hbm.at[pl.ds(base, BPW)], emb_v)
    lane = lax.iota(jnp.int32, 16)

    def batch_body(b, carry):
        def i_body(i, coll):
            row_i = [
                emb_v[b, i >> 1, pl.ds((i & 1) * D + q * 16, 16)]
                for q in range(NQ)
            ]
            p0 = i * (N - 1) - (i * (i - 1)) // 2 - i - 1

            def j_body(j, coll):
                rj = [
                    emb_v[b, j >> 1, pl.ds((j & 1) * D + q * 16, 16)]
                    for q in range(NQ)
                ]
                acc = row_i[0] * rj[0]
                for q in range(1, NQ):
                    acc = acc + row_i[q] * rj[q]
                p = p0 + j
                coll = jnp.where(lane == (p & 15), jnp.sum(acc), coll)
                out_v[b, pl.ds((p >> 4) * 16, 16)] = coll
                return coll

            return lax.fori_loop(i + 1, N, j_body, coll)

        return lax.fori_loop(0, N - 1, i_body, carry)

    lax.fori_loop(0, BPW, batch_body, jnp.zeros((16,), jnp.float32))
    pltpu.sync_copy(out_v, out_hbm.at[pl.ds(base, BPW)])


_edge_kernel = functools.partial(
    pl.kernel,
    out_type=jax.ShapeDtypeStruct((B, P), jnp.float32),
    mesh=plsc.VectorSubcoreMesh(core_axis_name="c", subcore_axis_name="s"),
    scratch_types=[
        pltpu.VMEM((BPW, N // 2, 2 * D), jnp.float32),
        pltpu.VMEM((BPW, P), jnp.float32),
    ],
    compiler_params=pltpu.CompilerParams(needs_layout_passes=False),
)(_edge_body)


def kernel(embeddings):
    i, j = jnp.triu_indices(N, k=1)
    node_combinations = jnp.stack([i, j], axis=1)
    edge_logits = _edge_kernel(embeddings.reshape(B, N // 2, 2 * D))
    return (edge_logits, node_combinations)


# speedup vs baseline: 3.9236x; 3.9236x over previous
import functools

import jax
import jax.numpy as jnp
import numpy as np
from jax import lax
from jax.experimental import pallas as pl
from jax.experimental.pallas import tpu as pltpu
from jax.experimental.pallas import tpu_sc as plsc

B, N, D = 1024, 32, 64
NW = 32
BPW = B // NW
P = N * (N - 1) // 2
NQ = D // 16
RB = 8


def _pflat(i, j):
    return i * (N - 1) - (i * (i - 1)) // 2 + (j - i - 1)


def _build_order():
    order = []
    for t in range(N // RB):
        i0 = t * RB
        for r in range(RB):
            for s in range(r + 1, RB):
                order.append(_pflat(i0 + r, i0 + s))
        for j0 in range(i0 + RB, N, 2):
            for j in (j0, j0 + 1):
                for r in range(RB):
                    order.append(_pflat(i0 + r, j))
    assert len(order) == P and sorted(order) == list(range(P))
    return np.asarray(order, dtype=np.int32)


_ORDER = _build_order()


def _load_row(emb_v, b, i):
    return [
        emb_v[b, i >> 1, pl.ds((i & 1) * D + q * 16, 16)] for q in range(NQ)
    ]


def _dot(ra, rb):
    m = [ra[q] * rb[q] for q in range(NQ)]
    return jnp.sum((m[0] + m[1]) + (m[2] + m[3]))


def _edge_body(emb_hbm, tab_hbm, out_hbm, emb_v, tab_v, out_v):
    wid = lax.axis_index("s") * 2 + lax.axis_index("c")
    base = wid * BPW
    pltpu.sync_copy(emb_hbm.at[pl.ds(base, BPW)], emb_v)
    pltpu.sync_copy(tab_hbm, tab_v)
    lane = lax.iota(jnp.int32, 16)

    def batch_body(b, coll):
        bvec = jnp.full((16,), b, jnp.int32)

        def flush(coll, g16):
            idx = tab_v[pl.ds(g16, 16)]
            plsc.store_scatter(out_v, [bvec, idx], coll)

        c = 0
        for t in range(N // RB):
            i0 = t * RB
            rows = [_load_row(emb_v, b, i0 + r) for r in range(RB)]
            for r in range(RB):
                for s in range(r + 1, RB):
                    tot = _dot(rows[r], rows[s])
                    coll = jnp.where(lane == (c & 15), tot, coll)
                    if (c & 15) == 15:
                        flush(coll, (c >> 4) * 16)
                    c += 1
            nj = N - (i0 + RB)
            if nj > 0:
                cb = c

                def m_body(m, coll, i0=i0, cb=cb):
                    jrow = (i0 + RB) // 2 + m
                    rj = [emb_v[b, jrow, pl.ds(q * 16, 16)] for q in range(2 * NQ)]
                    for u in range(16):
                        dj, r = u // 8, u % 8
                        tot = _dot(rows[r], rj[dj * NQ:(dj + 1) * NQ])
                        coll = jnp.where(lane == ((cb + u) & 15), tot, coll)
                        if ((cb + u) & 15) == 15:
                            flush(coll, ((cb + 16 * m + u) >> 4) * 16)
                    return coll

                coll = lax.fori_loop(0, nj // 2, m_body, coll)
                c += 8 * nj
        return coll

    lax.fori_loop(0, BPW, batch_body, jnp.zeros((16,), jnp.float32))
    pltpu.sync_copy(out_v, out_hbm.at[pl.ds(base, BPW)])


_edge_kernel = functools.partial(
    pl.kernel,
    out_type=jax.ShapeDtypeStruct((B, P), jnp.float32),
    mesh=plsc.VectorSubcoreMesh(core_axis_name="c", subcore_axis_name="s"),
    scratch_types=[
        pltpu.VMEM((BPW, N // 2, 2 * D), jnp.float32),
        pltpu.VMEM((P,), jnp.int32),
        pltpu.VMEM((BPW, P), jnp.float32),
    ],
    compiler_params=pltpu.CompilerParams(needs_layout_passes=False),
)(_edge_body)


def kernel(embeddings):
    i, j = jnp.triu_indices(N, k=1)
    node_combinations = jnp.stack([i, j], axis=1)
    edge_logits = _edge_kernel(
        embeddings.reshape(B, N // 2, 2 * D), jnp.asarray(_ORDER)
    )
    return (edge_logits, node_combinations)
